# Initial kernel scaffold; baseline (speedup 1.0000x reference)
import jax
import jax.numpy as jnp
from jax.experimental import pallas as pl


def kernel(X, H, W0, W2_0, W3_0, b0, a0, a2_0, ctx0, res0, g0, be0, al0, W1, W2_1, W3_1, b1, a1, a2_1, ctx1, res1, g1, be1, al1, cW1, cb1, cW2, cb2):
    raise NotImplementedError("write your pallas kernel here")


# speedup vs baseline: 1.4097x; 1.4097x over previous
import jax
import jax.numpy as jnp
from jax.experimental import pallas as pl

N, E = 10000, 2000
NEG = 0.2
RB = 1000
NRB = N // RB


def _lrelu(x):
    return jnp.where(x > 0, x, NEG * x)


def _prep_kernel(x_ref, W_ref, W2_ref, b_ref, ctx_ref, alo_ref, ahi_ref,
                 a2lo_ref, xt_ref, s1_ref, s2_ref):
    x = x_ref[...]
    x4 = jnp.dot(x, W2_ref[...], preferred_element_type=jnp.float32)
    xt_ref[...] = jnp.dot(x, W_ref[...],
                          preferred_element_type=jnp.float32) + b_ref[...]
    sctx = jnp.dot(ctx_ref[...], alo_ref[...],
                   preferred_element_type=jnp.float32)
    s1_ref[...] = _lrelu(sctx + jnp.dot(x4, ahi_ref[...],
                                        preferred_element_type=jnp.float32))
    s2_ref[...] = jnp.dot(x4, a2lo_ref[...],
                          preferred_element_type=jnp.float32)


def _edge_kernel(s1_ref, xt_ref, H_ref, Pn_ref, Pd_ref):
    k = pl.program_id(0)

    @pl.when(k == 0)
    def _():
        Pn_ref[...] = jnp.zeros_like(Pn_ref)
        Pd_ref[...] = jnp.zeros_like(Pd_ref)

    m1 = jnp.max(s1_ref[...])
    s1b = s1_ref[pl.ds(k * RB, RB), :]
    w = jnp.exp(s1b - m1)
    xt = xt_ref[...]
    H = H_ref[...]
    Pn_ref[...] += jax.lax.dot_general(
        H, xt * w, (((0,), (0,)), ((), ())),
        preferred_element_type=jnp.float32)
    Pd_ref[...] += jax.lax.dot_general(
        H, jnp.broadcast_to(w, (RB, 8)), (((0,), (0,)), ((), ())),
        preferred_element_type=jnp.float32)


def _edgepost_kernel(Pn_ref, Pd_ref, W3_ref, a2hi_ref, edge_ref, s3_ref):
    edge = Pn_ref[...] / Pd_ref[:, 0:1]
    edge_ref[...] = edge
    e4 = jnp.dot(edge, W3_ref[...], preferred_element_type=jnp.float32)
    s3_ref[...] = jax.lax.dot_general(
        a2hi_ref[...], e4, (((0,), (1,)), ((), ())),
        preferred_element_type=jnp.float32)


def _node_kernel(H_ref, s2_ref, x_ref, edge_ref, s3_ref, res_ref, g_ref,
                 be_ref, al_ref, out_ref):
    s3 = s3_ref[...]
    m3 = jnp.max(s3)
    s2 = s2_ref[...]
    S = _lrelu(s2 + s3)
    mrow = _lrelu(s2 + m3)
    A = H_ref[...] * jnp.exp(S - mrow)
    num = jnp.dot(A, edge_ref[...], preferred_element_type=jnp.float32)
    den = jnp.sum(A, axis=1, keepdims=True)
    node = num / den
    y = jnp.where(node > 0, node, jnp.exp(jnp.minimum(node, 0.0)) - 1.0)
    mu = jnp.mean(y, axis=1, keepdims=True)
    c = y - mu
    v = jnp.mean(c * c, axis=1, keepdims=True)
    xn = c * jax.lax.rsqrt(v + 1e-5) * g_ref[...] + be_ref[...]
    al = al_ref[0, 0]
    out_ref[...] = al * xn + (1.0 - al) * jnp.dot(
        x_ref[...], res_ref[...], preferred_element_type=jnp.float32)


def _cls_kernel(x1_ref, x2_ref, cW1a_ref, cW1b_ref, cb1_ref, cW2_ref,
                cb2_ref, out_ref):
    h = (jnp.dot(x1_ref[...], cW1a_ref[...],
                 preferred_element_type=jnp.float32)
         + jnp.dot(x2_ref[...], cW1b_ref[...],
                   preferred_element_type=jnp.float32)
         + cb1_ref[...])
    h = jnp.maximum(h, 0.0)
    out_ref[...] = jnp.dot(h, cW2_ref[...],
                           preferred_element_type=jnp.float32) + cb2_ref[...]


def _full(shape):
    nd = len(shape)
    return pl.BlockSpec(shape, lambda i: (0,) * nd)


def _rows(shape):
    nd = len(shape)
    return pl.BlockSpec((RB,) + shape[1:], lambda i: (i,) + (0,) * (nd - 1))


def _layer(x, H, W, W2, W3, b, a, a2, ctx, res, g, be, al):
    di, do = W.shape
    f32 = jnp.float32
    b2 = b.reshape(1, do)
    ctx2 = ctx.reshape(1, do)
    alo, ahi = a[:do], a[do:]
    a2lo, a2hi = a2[:do], a2[do:]
    g2, be2 = g.reshape(1, do), be.reshape(1, do)
    al2 = al.reshape(1, 1)

    xt, s1, s2 = pl.pallas_call(
        _prep_kernel,
        grid=(NRB,),
        in_specs=[_rows((RB, di)), _full((di, do)), _full((di, do)),
                  _full((1, do)), _full((1, do)), _full((do, 1)),
                  _full((do, 1)), _full((do, 1))],
        out_specs=[_rows((RB, do)), _rows((RB, 1)), _rows((RB, 1))],
        out_shape=[jax.ShapeDtypeStruct((N, do), f32),
                   jax.ShapeDtypeStruct((N, 1), f32),
                   jax.ShapeDtypeStruct((N, 1), f32)],
    )(x, W, W2, b2, ctx2, alo, ahi, a2lo)

    Pn, Pd = pl.pallas_call(
        _edge_kernel,
        grid=(NRB,),
        in_specs=[_full((N, 1)), _rows((RB, do)), _rows((RB, E))],
        out_specs=[_full((E, do)), _full((E, 8))],
        out_shape=[jax.ShapeDtypeStruct((E, do), f32),
                   jax.ShapeDtypeStruct((E, 8), f32)],
    )(s1, xt, H)

    edge, s3 = pl.pallas_call(
        _edgepost_kernel,
        grid=(1,),
        in_specs=[_full((E, do)), _full((E, 8)), _full((do, do)),
                  _full((do, 1))],
        out_specs=[_full((E, do)), _full((1, E))],
        out_shape=[jax.ShapeDtypeStruct((E, do), f32),
                   jax.ShapeDtypeStruct((1, E), f32)],
    )(Pn, Pd, W3, a2hi)

    x_new = pl.pallas_call(
        _node_kernel,
        grid=(NRB,),
        in_specs=[_rows((RB, E)), _rows((RB, 1)), _rows((RB, di)),
                  _full((E, do)), _full((1, E)), _full((di, do)),
                  _full((1, do)), _full((1, do)), _full((1, 1))],
        out_specs=_rows((RB, do)),
        out_shape=jax.ShapeDtypeStruct((N, do), f32),
    )(H, s2, x, edge, s3, res, g2, be2, al2)
    return x_new


def kernel(X, H, W0, W2_0, W3_0, b0, a0, a2_0, ctx0, res0, g0, be0, al0,
           W1, W2_1, W3_1, b1, a1, a2_1, ctx1, res1, g1, be1, al1,
           cW1, cb1, cW2, cb2):
    x1 = _layer(X, H, W0, W2_0, W3_0, b0, a0, a2_0, ctx0, res0, g0, be0, al0)
    x2 = _layer(x1, H, W1, W2_1, W3_1, b1, a1, a2_1, ctx1, res1, g1, be1, al1)

    h1 = x1.shape[1]
    out_dim = cW2.shape[1]
    hid = cW1.shape[1]
    cW1a, cW1b = cW1[:h1], cW1[h1:]
    out = pl.pallas_call(
        _cls_kernel,
        grid=(NRB,),
        in_specs=[_rows((RB, h1)), _rows((RB, x2.shape[1])),
                  _full((h1, hid)), _full((x2.shape[1], hid)),
                  _full((1, hid)), _full((hid, out_dim)),
                  _full((1, out_dim))],
        out_specs=_rows((RB, out_dim)),
        out_shape=jax.ShapeDtypeStruct((N, out_dim), jnp.float32),
    )(x1, x2, cW1a, cW1b, cb1.reshape(1, hid), cW2,
      cb2.reshape(1, out_dim))
    return out


# speedup vs baseline: 1.4138x; 1.0029x over previous
import jax
import jax.numpy as jnp
from jax.experimental import pallas as pl
from jax.experimental.pallas import tpu as pltpu

N, E = 10000, 2000
NEG = 0.2
RB = 400
NRB = N // RB
BF = jnp.bfloat16
F32 = jnp.float32


def _lrelu(x):
    return jnp.where(x > 0, x, NEG * x)


def _dotT(a, b):
    return jax.lax.dot_general(a, b, (((0,), (0,)), ((), ())),
                               preferred_element_type=F32)


def _edge_body(x_ref, W_ref, W2_ref, b_ref, ctx_ref, alo_ref, ahi_ref,
               a2lo_ref, Pn_ref, Pd_ref, s2_ref, m_ref, Hb):
    k = pl.program_id(0)

    @pl.when(k == 0)
    def _():
        Pn_ref[...] = jnp.zeros_like(Pn_ref)
        Pd_ref[...] = jnp.zeros_like(Pd_ref)
        m_ref[0, 0] = -1e30

    x = x_ref[...]
    x4 = jnp.dot(x, W2_ref[...], preferred_element_type=F32)
    xt = jnp.dot(x, W_ref[...], preferred_element_type=F32) + b_ref[...]
    sctx = jnp.dot(ctx_ref[...], alo_ref[...], preferred_element_type=F32)
    s1 = _lrelu(sctx + jnp.dot(x4, ahi_ref[...],
                               preferred_element_type=F32))
    s2_ref[...] = jnp.dot(x4, a2lo_ref[...], preferred_element_type=F32)

    m_old = m_ref[0, 0]
    m_new = jnp.maximum(m_old, jnp.max(s1))
    alpha = jnp.exp(m_old - m_new)
    w = jnp.exp(s1 - m_new)
    Pn_ref[...] = alpha * Pn_ref[...] + _dotT(Hb, (xt * w).astype(BF))
    Pd_ref[...] = alpha * Pd_ref[...] + _dotT(
        Hb, jnp.broadcast_to(w, (w.shape[0], 8)).astype(BF))
    m_ref[0, 0] = m_new


def _edge0_kernel(x_ref, H_ref, W_ref, W2_ref, b_ref, ctx_ref, alo_ref,
                  ahi_ref, a2lo_ref, Pn_ref, Pd_ref, s2_ref, Hb_ref, m_ref):
    Hb = H_ref[...].astype(BF)
    Hb_ref[...] = Hb
    _edge_body(x_ref, W_ref, W2_ref, b_ref, ctx_ref, alo_ref, ahi_ref,
               a2lo_ref, Pn_ref, Pd_ref, s2_ref, m_ref, Hb)


def _edge1_kernel(x_ref, H_ref, W_ref, W2_ref, b_ref, ctx_ref, alo_ref,
                  ahi_ref, a2lo_ref, Pn_ref, Pd_ref, s2_ref, m_ref):
    _edge_body(x_ref, W_ref, W2_ref, b_ref, ctx_ref, alo_ref, ahi_ref,
               a2lo_ref, Pn_ref, Pd_ref, s2_ref, m_ref, H_ref[...])


def _node_body(H_ref, s2_ref, x_ref, Pn_ref, Pd_ref, W3_ref, a2hi_ref,
               res_ref, g_ref, be_ref, al_ref, eb_ref, s3_ref):
    k = pl.program_id(0)

    @pl.when(k == 0)
    def _():
        edge = Pn_ref[...] / Pd_ref[:, 0:1]
        eb_ref[...] = edge.astype(BF)
        e4 = jnp.dot(edge, W3_ref[...], preferred_element_type=F32)
        s3_ref[...] = jax.lax.dot_general(
            a2hi_ref[...], e4, (((0,), (1,)), ((), ())),
            preferred_element_type=F32)

    s3 = s3_ref[...]
    m3 = jnp.max(s3)
    s2 = s2_ref[...]
    S = _lrelu(s2 + s3)
    mrow = _lrelu(s2 + m3)
    A = H_ref[...] * jnp.exp(S - mrow).astype(BF)
    eb = eb_ref[...]
    num = jnp.dot(A, eb, preferred_element_type=F32)
    den = jax.lax.dot_general(
        A, jnp.ones((E, 8), BF), (((1,), (0,)), ((), ())),
        preferred_element_type=F32)[:, 0:1]
    node = num / den
    y = jnp.where(node > 0, node, jnp.exp(jnp.minimum(node, 0.0)) - 1.0)
    mu = jnp.mean(y, axis=1, keepdims=True)
    c = y - mu
    v = jnp.mean(c * c, axis=1, keepdims=True)
    xn = c * jax.lax.rsqrt(v + 1e-5) * g_ref[...] + be_ref[...]
    al = al_ref[0, 0]
    return al * xn + (1.0 - al) * jnp.dot(
        x_ref[...], res_ref[...], preferred_element_type=F32)


def _node0_kernel(H_ref, s2_ref, x_ref, Pn_ref, Pd_ref, W3_ref, a2hi_ref,
                  res_ref, g_ref, be_ref, al_ref, out_ref, eb_ref, s3_ref):
    out_ref[...] = _node_body(H_ref, s2_ref, x_ref, Pn_ref, Pd_ref, W3_ref,
                              a2hi_ref, res_ref, g_ref, be_ref, al_ref,
                              eb_ref, s3_ref)


def _node1_kernel(H_ref, s2_ref, x_ref, Pn_ref, Pd_ref, W3_ref, a2hi_ref,
                  res_ref, g_ref, be_ref, al_ref, cW1a_ref, cW1b_ref,
                  cb1_ref, cW2_ref, cb2_ref, out_ref, eb_ref, s3_ref):
    x2 = _node_body(H_ref, s2_ref, x_ref, Pn_ref, Pd_ref, W3_ref, a2hi_ref,
                    res_ref, g_ref, be_ref, al_ref, eb_ref, s3_ref)
    h = (jnp.dot(x_ref[...], cW1a_ref[...], preferred_element_type=F32)
         + jnp.dot(x2, cW1b_ref[...], preferred_element_type=F32)
         + cb1_ref[...])
    h = jnp.maximum(h, 0.0)
    out_ref[...] = jnp.dot(h, cW2_ref[...],
                           preferred_element_type=F32) + cb2_ref[...]


def _full(shape):
    nd = len(shape)
    return pl.BlockSpec(shape, lambda i: (0,) * nd)


def _rows(shape):
    nd = len(shape)
    return pl.BlockSpec((RB,) + shape[1:], lambda i: (i,) + (0,) * (nd - 1))


def _edge_call(lay0, x, H, W, W2, b, ctx, alo, ahi, a2lo):
    di, do = W.shape
    Hdt = F32 if lay0 else BF
    out_shape = [jax.ShapeDtypeStruct((E, do), F32),
                 jax.ShapeDtypeStruct((E, 8), F32),
                 jax.ShapeDtypeStruct((N, 1), F32)]
    out_specs = [_full((E, do)), _full((E, 8)), _rows((RB, 1))]
    if lay0:
        out_shape.append(jax.ShapeDtypeStruct((N, E), BF))
        out_specs.append(_rows((RB, E)))
    return pl.pallas_call(
        _edge0_kernel if lay0 else _edge1_kernel,
        grid=(NRB,),
        in_specs=[_rows((RB, di)), _rows((RB, E)), _full((di, do)),
                  _full((di, do)), _full((1, do)), _full((1, do)),
                  _full((do, 1)), _full((do, 1)), _full((do, 1))],
        out_specs=out_specs,
        out_shape=out_shape,
        scratch_shapes=[pltpu.SMEM((1, 1), F32)],
    )(x, H, W, W2, b.reshape(1, do), ctx.reshape(1, do), alo, ahi, a2lo)


def kernel(X, H, W0, W2_0, W3_0, b0, a0, a2_0, ctx0, res0, g0, be0, al0,
           W1, W2_1, W3_1, b1, a1, a2_1, ctx1, res1, g1, be1, al1,
           cW1, cb1, cW2, cb2):
    do0 = W0.shape[1]
    do1 = W1.shape[1]

    Pn0, Pd0, s2_0, Hb = _edge_call(
        True, X, H, W0, W2_0, b0, ctx0, a0[:do0], a0[do0:], a2_0[:do0])

    x1 = pl.pallas_call(
        _node0_kernel,
        grid=(NRB,),
        in_specs=[_rows((RB, E)), _rows((RB, 1)), _rows((RB, X.shape[1])),
                  _full((E, do0)), _full((E, 8)), _full((do0, do0)),
                  _full((do0, 1)), _full((X.shape[1], do0)),
                  _full((1, do0)), _full((1, do0)), _full((1, 1))],
        out_specs=_rows((RB, do0)),
        out_shape=jax.ShapeDtypeStruct((N, do0), F32),
        scratch_shapes=[pltpu.VMEM((E, do0), BF), pltpu.VMEM((1, E), F32)],
    )(Hb, s2_0, X, Pn0, Pd0, W3_0, a2_0[do0:], res0,
      g0.reshape(1, do0), be0.reshape(1, do0), al0.reshape(1, 1))

    Pn1, Pd1, s2_1 = _edge_call(
        False, x1, Hb, W1, W2_1, b1, ctx1, a1[:do1], a1[do1:], a2_1[:do1])

    hid = cW1.shape[1]
    odim = cW2.shape[1]
    out = pl.pallas_call(
        _node1_kernel,
        grid=(NRB,),
        in_specs=[_rows((RB, E)), _rows((RB, 1)), _rows((RB, do0)),
                  _full((E, do1)), _full((E, 8)), _full((do1, do1)),
                  _full((do1, 1)), _full((do0, do1)),
                  _full((1, do1)), _full((1, do1)), _full((1, 1)),
                  _full((do0, hid)), _full((do1, hid)), _full((1, hid)),
                  _full((hid, odim)), _full((1, odim))],
        out_specs=_rows((RB, odim)),
        out_shape=jax.ShapeDtypeStruct((N, odim), F32),
        scratch_shapes=[pltpu.VMEM((E, do1), BF), pltpu.VMEM((1, E), F32)],
    )(Hb, s2_1, x1, Pn1, Pd1, W3_1, a2_1[do1:], res1,
      g1.reshape(1, do1), be1.reshape(1, do1), al1.reshape(1, 1),
      cW1[:do0], cW1[do0:], cb1.reshape(1, hid), cW2,
      cb2.reshape(1, odim))
    return out
